# Initial kernel scaffold; baseline (speedup 1.0000x reference)
import jax
import jax.numpy as jnp
from jax.experimental import pallas as pl


def kernel(x, edge_index, W1_l, W1_r, b1, W2_l, W2_r, b2):
    raise NotImplementedError("write your pallas kernel here")


# speedup vs baseline: 7.9046x; 7.9046x over previous
import functools

import jax
import jax.numpy as jnp
from jax import lax
from jax.experimental import pallas as pl
from jax.experimental.pallas import tpu as pltpu
from jax.experimental.pallas import tpu_sc as plsc

N = 10000
NP = 10240
E = 320000
D = 128
NC = 2
NS = 16
NW = NC * NS
EPW = E // NW
CH = 80
NCHUNK = EPW // CH
RPS = NP // NS
BLK = 1024
GRID = NP // BLK

_MESH = plsc.VectorSubcoreMesh(core_axis_name="c", subcore_axis_name="s")


def _make_seg_kernel(with_cnt: bool):
    out_type = [jax.ShapeDtypeStruct((NC, NP, D), jnp.float32)]
    if with_cnt:
        out_type.append(jax.ShapeDtypeStruct((NC, NP), jnp.float32))

    scratch = [
        pltpu.VMEM((NCHUNK, CH), jnp.int32),
        pltpu.VMEM((NCHUNK, CH), jnp.int32),
        pltpu.VMEM((CH, D), jnp.float32),
        pltpu.VMEM((CH,), jnp.float32),
        pltpu.VMEM((RPS,), jnp.float32),
        pltpu.VMEM_SHARED((NP, D), jnp.float32),
        pltpu.VMEM_SHARED((NP,), jnp.float32),
        pltpu.SemaphoreType.DMA,
    ]

    @functools.partial(pl.kernel, mesh=_MESH, out_type=out_type,
                       scratch_types=scratch)
    def seg(y_hbm, src_hbm, dst_hbm, *refs):
        if with_cnt:
            sum_hbm, cnt_hbm = refs[0], refs[1]
            rest = refs[2:]
        else:
            sum_hbm = refs[0]
            cnt_hbm = None
            rest = refs[1:]
        (src_v, dst_v, rows_v, ones_v, zcnt_v,
         acc_sh, cnt_sh, sem) = rest

        cid = lax.axis_index("c")
        sid = lax.axis_index("s")
        wid = sid * NC + cid

        def zfill(i, carry):
            for k in range(D // 16):
                rows_v[i, pl.ds(k * 16, 16)] = jnp.zeros((16,), jnp.float32)
            return carry
        lax.fori_loop(0, CH, zfill, 0)

        def zcfill(i, carry):
            zcnt_v[pl.ds(i * 16, 16)] = jnp.zeros((16,), jnp.float32)
            return carry
        lax.fori_loop(0, RPS // 16, zcfill, 0)
        for k in range(CH // 16):
            ones_v[pl.ds(k * 16, 16)] = jnp.ones((16,), jnp.float32)

        base = sid * RPS
        for t in range(RPS // CH):
            pltpu.sync_copy(rows_v, acc_sh.at[pl.ds(base + t * CH, CH)])
        pltpu.sync_copy(zcnt_v, cnt_sh.at[pl.ds(base, RPS)])
        plsc.subcore_barrier()

        pltpu.sync_copy(src_hbm.at[wid], src_v)
        pltpu.sync_copy(dst_hbm.at[wid], dst_v)

        def step(j, carry):
            pltpu.async_copy(y_hbm.at[src_v.at[j]], rows_v, sem).wait()
            pltpu.sync_copy(rows_v, acc_sh.at[dst_v.at[j]], add=True)
            if with_cnt:
                pltpu.sync_copy(ones_v, cnt_sh.at[dst_v.at[j]], add=True)
            return carry
        lax.fori_loop(0, NCHUNK, step, 0)

        plsc.subcore_barrier()

        pltpu.sync_copy(acc_sh.at[pl.ds(base, RPS)],
                        sum_hbm.at[cid, pl.ds(base, RPS)])
        if with_cnt:
            pltpu.sync_copy(cnt_sh.at[pl.ds(base, RPS)],
                            cnt_hbm.at[cid, pl.ds(base, RPS)])

    return seg


_seg_with_cnt = _make_seg_kernel(True)
_seg_no_cnt = _make_seg_kernel(False)


def _lin1_body(x_ref, s_ref, c_ref, wl_ref, wr_ref, b_ref, h_ref, cnt_ref):
    cnt = jnp.maximum(c_ref[0] + c_ref[1], 1.0)
    agg = (s_ref[0] + s_ref[1]) / cnt[:, None]
    h = (jnp.dot(agg, wl_ref[...], preferred_element_type=jnp.float32)
         + jnp.dot(x_ref[...], wr_ref[...], preferred_element_type=jnp.float32)
         + b_ref[...][None, :])
    h_ref[...] = jnp.maximum(h, 0.0)
    cnt_ref[...] = cnt


def _lin2_body(x_ref, s_ref, cnt_ref, wl_ref, wr_ref, b_ref, o_ref):
    agg = (s_ref[0] + s_ref[1]) / cnt_ref[...][:, None]
    o_ref[...] = (jnp.dot(agg, wl_ref[...], preferred_element_type=jnp.float32)
                  + jnp.dot(x_ref[...], wr_ref[...],
                            preferred_element_type=jnp.float32)
                  + b_ref[...][None, :])


def _lin1(xp, s1p, c1p, WlT, WrT, b):
    return pl.pallas_call(
        _lin1_body,
        grid=(GRID,),
        in_specs=[
            pl.BlockSpec((BLK, D), lambda i: (i, 0)),
            pl.BlockSpec((NC, BLK, D), lambda i: (0, i, 0)),
            pl.BlockSpec((NC, BLK), lambda i: (0, i)),
            pl.BlockSpec((D, D), lambda i: (0, 0)),
            pl.BlockSpec((D, D), lambda i: (0, 0)),
            pl.BlockSpec((D,), lambda i: (0,)),
        ],
        out_specs=[
            pl.BlockSpec((BLK, D), lambda i: (i, 0)),
            pl.BlockSpec((BLK,), lambda i: (i,)),
        ],
        out_shape=[
            jax.ShapeDtypeStruct((NP, D), jnp.float32),
            jax.ShapeDtypeStruct((NP,), jnp.float32),
        ],
    )(xp, s1p, c1p, WlT, WrT, b)


def _lin2(h, s2p, cnt, WlT, WrT, b):
    return pl.pallas_call(
        _lin2_body,
        grid=(GRID,),
        in_specs=[
            pl.BlockSpec((BLK, D), lambda i: (i, 0)),
            pl.BlockSpec((NC, BLK, D), lambda i: (0, i, 0)),
            pl.BlockSpec((BLK,), lambda i: (i,)),
            pl.BlockSpec((D, D), lambda i: (0, 0)),
            pl.BlockSpec((D, D), lambda i: (0, 0)),
            pl.BlockSpec((D,), lambda i: (0,)),
        ],
        out_specs=pl.BlockSpec((BLK, D), lambda i: (i, 0)),
        out_shape=jax.ShapeDtypeStruct((NP, D), jnp.float32),
    )(h, s2p, cnt, WlT, WrT, b)


def kernel(x, edge_index, W1_l, W1_r, b1, W2_l, W2_r, b2):
    xp = jnp.zeros((NP, D), jnp.float32).at[:N].set(x)
    src3 = edge_index[0].reshape(NW, NCHUNK, CH)
    dst3 = edge_index[1].reshape(NW, NCHUNK, CH)

    s1p, c1p = _seg_with_cnt(xp, src3, dst3)
    h, cnt = _lin1(xp, s1p, c1p, W1_l.T, W1_r.T, b1)
    (s2p,) = _seg_no_cnt(h, src3, dst3)
    out = _lin2(h, s2p, cnt, W2_l.T, W2_r.T, b2)
    return out[:N]
